# baseline (speedup 1.0000x reference)
import functools

import jax
import jax.numpy as jnp
from jax import lax
from jax.experimental import pallas as pl
from jax.experimental.pallas import tpu as pltpu
from jax.experimental.pallas import tpu_sc as plsc

N = 10000
E = 320000
H = 128
HH = 64
A_DIM = 8
MLP_H = 64

NB = 10
NBS = N // NB
EBLK = 80
EBS = E // EBLK

NC = 2
NT = 16
K = 40
CPT = E // (K * NT)

EPS_DEN = 1e-6
EPS_BN = 1e-5

_F32 = jnp.float32


def _dot(a, b):
    return jnp.dot(a, b, preferred_element_type=_F32)


def _node_pre_body(h_ref, wh, a1, b1, e1w, d1, h0_o, st_o, dt_o, hd_o):
    h0 = _dot(h_ref[...], wh[...])
    h0_o[...] = h0
    ha = _dot(h0, a1[...])
    hv = _dot(h0, e1w[...])
    st_o[0] = jnp.concatenate([ha[:, :HH], hv[:, :HH]], axis=1)
    st_o[1] = jnp.concatenate([ha[:, HH:], hv[:, HH:]], axis=1)
    dt_o[...] = _dot(h0, b1[...])
    hd_o[...] = _dot(h0, d1[...])


def _edge_lin_body(e_ref, we_ref, c_ref, out_ref):
    wec = _dot(we_ref[...], c_ref[...])
    out_ref[...] = _dot(e_ref[...], wec)


def _upd_stats_body(hd_ref, nd_ref, hn_o, st_o):
    num = jnp.concatenate([nd_ref[0, :, :HH], nd_ref[1, :, :HH]], axis=1)
    den = jnp.concatenate([nd_ref[0, :, HH:], nd_ref[1, :, HH:]], axis=1)
    hn = hd_ref[...] + num / (den + EPS_DEN)
    hn_o[...] = hn

    @pl.when(pl.program_id(0) == 0)
    def _():
        st_o[...] = jnp.zeros_like(st_o)

    part = jnp.concatenate(
        [jnp.sum(hn, 0, keepdims=True), jnp.sum(hn * hn, 0, keepdims=True)], 0)
    st_o[0:2, :] += part


def _h1_pre2_body(hres_ref, hn_ref, st_ref, a2, b2, e2w, d2,
                  h1_o, st_o, dt_o, hd_o):
    m = st_ref[0:1, :] / N
    var = st_ref[1:2, :] / N - m * m
    rstd = lax.rsqrt(var + EPS_BN)
    h1 = hres_ref[...] + jnp.maximum((hn_ref[...] - m) * rstd, 0.0)
    h1_o[...] = h1
    ha = _dot(h1, a2[...])
    hv = _dot(h1, e2w[...])
    st_o[0] = jnp.concatenate([ha[:, :HH], hv[:, :HH]], axis=1)
    st_o[1] = jnp.concatenate([ha[:, HH:], hv[:, HH:]], axis=1)
    dt_o[...] = _dot(h1, b2[...])
    hd_o[...] = _dot(h1, d2[...])


def _estats_body(en_ref, st_o):
    x = jnp.concatenate([en_ref[0][:, :HH], en_ref[1][:, HH:]], axis=1)
    s = jnp.sum(x, 0, keepdims=True)
    q = jnp.sum(x * x, 0, keepdims=True)

    @pl.when(pl.program_id(0) == 0)
    def _():
        st_o[...] = jnp.zeros_like(st_o)

    st_o[0:2, :] += jnp.concatenate([s, q], 0)


def _edge2_body(en_ref, e_ref, st_ref, we_ref, c2_ref, out_ref):
    m = st_ref[0:1, :] / E
    var = st_ref[1:2, :] / E - m * m
    rstd = lax.rsqrt(var + EPS_BN)
    en = jnp.concatenate([en_ref[0][:, :HH], en_ref[1][:, HH:]], axis=1)
    r = jnp.maximum((en - m) * rstd, 0.0)
    wec2 = _dot(we_ref[...], c2_ref[...])
    out_ref[...] = _dot(r, c2_ref[...]) + _dot(e_ref[...], wec2)


def _final_body(h1_ref, hn_ref, st_ref, a_ref, wm1, bm1, wm2t, bm2,
                out_ref, acc):
    m = st_ref[0:1, :] / N
    var = st_ref[1:2, :] / N - m * m
    rstd = lax.rsqrt(var + EPS_BN)
    h2 = h1_ref[...] + jnp.maximum((hn_ref[...] - m) * rstd, 0.0)
    cs = jnp.sum(h2, 0, keepdims=True)
    pid = pl.program_id(0)

    @pl.when(pid == 0)
    def _():
        acc[...] = jnp.zeros_like(acc)

    acc[0:1, :] += cs

    @pl.when(pid == NB - 1)
    def _():
        hg = acc[0:1, :] / N
        z = jnp.concatenate([hg, a_ref[...]], axis=1)
        q = jnp.maximum(_dot(z, wm1[...]) + bm1[...], 0.0)
        out_ref[...] = jnp.sum(q * wm2t[...], axis=1, keepdims=True) + bm2[...]


def _full(shape):
    return pl.BlockSpec(shape, lambda i: tuple(0 for _ in shape))


def _node_pre(h, Wh, A1, B1, E1, D1):
    row = pl.BlockSpec((NBS, H), lambda i: (i, 0))
    srow = pl.BlockSpec((NC, NBS, H), lambda i: (0, i, 0))
    w = _full((H, H))
    return pl.pallas_call(
        _node_pre_body,
        grid=(NB,),
        in_specs=[row, w, w, w, w, w],
        out_specs=[row, srow, row, row],
        out_shape=[jax.ShapeDtypeStruct((N, H), _F32),
                   jax.ShapeDtypeStruct((NC, N, H), _F32),
                   jax.ShapeDtypeStruct((N, H), _F32),
                   jax.ShapeDtypeStruct((N, H), _F32)],
    )(h, Wh, A1, B1, E1, D1)


def _edge_lin(e, We, C1):
    return pl.pallas_call(
        _edge_lin_body,
        grid=(EBLK,),
        in_specs=[pl.BlockSpec((EBS, 16), lambda i: (i, 0)),
                  _full((16, H)), _full((H, H))],
        out_specs=pl.BlockSpec((EBS, H), lambda i: (i, 0)),
        out_shape=jax.ShapeDtypeStruct((E, H), _F32),
    )(e, We, C1)


def _upd_stats(hd, nd):
    row = pl.BlockSpec((NBS, H), lambda i: (i, 0))
    return pl.pallas_call(
        _upd_stats_body,
        grid=(NB,),
        in_specs=[row, pl.BlockSpec((NC, NBS, H), lambda i: (0, i, 0))],
        out_specs=[row, _full((8, H))],
        out_shape=[jax.ShapeDtypeStruct((N, H), _F32),
                   jax.ShapeDtypeStruct((8, H), _F32)],
    )(hd, nd)


def _h1_pre2(h0, hn1, st1, A2, B2, E2, D2):
    row = pl.BlockSpec((NBS, H), lambda i: (i, 0))
    srow = pl.BlockSpec((NC, NBS, H), lambda i: (0, i, 0))
    w = _full((H, H))
    return pl.pallas_call(
        _h1_pre2_body,
        grid=(NB,),
        in_specs=[row, row, _full((8, H)), w, w, w, w],
        out_specs=[row, srow, row, row],
        out_shape=[jax.ShapeDtypeStruct((N, H), _F32),
                   jax.ShapeDtypeStruct((NC, N, H), _F32),
                   jax.ShapeDtypeStruct((N, H), _F32),
                   jax.ShapeDtypeStruct((N, H), _F32)],
    )(h0, hn1, st1, A2, B2, E2, D2)


def _estats(enew):
    return pl.pallas_call(
        _estats_body,
        grid=(EBLK,),
        in_specs=[pl.BlockSpec((NC, EBS, H), lambda i: (0, i, 0))],
        out_specs=_full((8, H)),
        out_shape=jax.ShapeDtypeStruct((8, H), _F32),
    )(enew)


def _edge2(enew, e, est, We, C2):
    return pl.pallas_call(
        _edge2_body,
        grid=(EBLK,),
        in_specs=[pl.BlockSpec((NC, EBS, H), lambda i: (0, i, 0)),
                  pl.BlockSpec((EBS, 16), lambda i: (i, 0)),
                  _full((8, H)), _full((16, H)), _full((H, H))],
        out_specs=pl.BlockSpec((EBS, H), lambda i: (i, 0)),
        out_shape=jax.ShapeDtypeStruct((E, H), _F32),
    )(enew, e, est, We, C2)


def _final(h1, hn2, st2, a, Wm1, bm1, Wm2, bm2):
    row = pl.BlockSpec((NBS, H), lambda i: (i, 0))
    return pl.pallas_call(
        _final_body,
        grid=(NB,),
        in_specs=[row, row, _full((8, H)), _full((1, A_DIM)),
                  _full((H + A_DIM, MLP_H)), _full((1, MLP_H)),
                  _full((1, MLP_H)), _full((1, 1))],
        out_specs=_full((1, 1)),
        out_shape=jax.ShapeDtypeStruct((1, 1), _F32),
        scratch_shapes=[pltpu.VMEM((8, H), _F32)],
    )(h1, hn2, st2, a, Wm1, bm1, Wm2, bm2)


def _make_edge_pass(write_enew):
    mesh = plsc.VectorSubcoreMesh(core_axis_name="c", subcore_axis_name="s",
                                  num_cores=NC, num_subcores=NT)
    outs = [jax.ShapeDtypeStruct((NC, N, H), _F32)]
    if write_enew:
        outs.append(jax.ShapeDtypeStruct((NC, E, H), _F32))
    scratch = [
        pltpu.VMEM((K,), jnp.int32),
        pltpu.VMEM((K,), jnp.int32),
        pltpu.VMEM((K,), jnp.int32),
        pltpu.VMEM((K,), jnp.int32),
        pltpu.VMEM((K, H), _F32),
        pltpu.VMEM((K, H), _F32),
        pltpu.VMEM((K, H), _F32),
        pltpu.VMEM((K, H), _F32),
        pltpu.VMEM((K, H), _F32),
        pltpu.VMEM((K, H), _F32),
        pltpu.VMEM((K, H), _F32),
        pltpu.VMEM((K, H), _F32),
        pltpu.VMEM((K, H), _F32),
        pltpu.SemaphoreType.DMA,
        pltpu.SemaphoreType.DMA,
        pltpu.SemaphoreType.DMA,
        pltpu.SemaphoreType.DMA,
        pltpu.SemaphoreType.DMA,
        pltpu.SemaphoreType.DMA,
        pltpu.VMEM_SHARED((N, H), _F32),
    ]

    @functools.partial(pl.kernel, mesh=mesh,
                       out_type=tuple(outs) if write_enew else outs[0],
                       scratch_types=scratch)
    def edge_pass(sidx_hbm, didx_hbm, stab, dtab, ec, zb, *refs):
        if write_enew:
            nd_out, en_out = refs[0], refs[1]
            scr = refs[2:]
        else:
            nd_out = refs[0]
            en_out = None
            scr = refs[1:]
        (si0, di0, si1, di1, sr0, sr1, dr0, dr1, eb0, eb1, outb, en0, en1,
         semi0, semi1, semg0, semg1, semw0, semw1, acc) = scr
        sib, dib = (si0, si1), (di0, di1)
        srb, drb, ecbb = (sr0, sr1), (dr0, dr1), (eb0, eb1)
        enbb = (en0, en1)
        semi, semg, semw = (semi0, semi1), (semg0, semg1), (semw0, semw1)
        c = lax.axis_index("c")
        s = lax.axis_index("s")
        tbase = s * (CPT * K)

        def issue_idx(i, b):
            base = tbase + i * K
            pltpu.async_copy(sidx_hbm.at[pl.ds(base, K)], sib[b], semi[b])
            pltpu.async_copy(didx_hbm.at[pl.ds(base, K)], dib[b], semi[b])

        def wait_idx(b):
            pltpu.make_async_copy(sidx_hbm.at[pl.ds(0, K)], sib[b],
                                  semi[b]).wait()
            pltpu.make_async_copy(didx_hbm.at[pl.ds(0, K)], dib[b],
                                  semi[b]).wait()

        def issue_gather(i, b):
            base = tbase + i * K
            pltpu.async_copy(stab.at[c].at[sib[b]], srb[b], semg[b])
            pltpu.async_copy(dtab.at[dib[b]], drb[b], semg[b])
            pltpu.async_copy(ec.at[pl.ds(base, K), :], ecbb[b], semg[b])

        def wait_gather(b):
            pltpu.make_async_copy(stab.at[c].at[sib[b]], srb[b],
                                  semg[b]).wait()
            pltpu.make_async_copy(dtab.at[dib[b]], drb[b],
                                  semg[b]).wait()
            pltpu.make_async_copy(ec.at[pl.ds(0, K), :], ecbb[b],
                                  semg[b]).wait()

        def wait_enew(b):
            pltpu.make_async_copy(enbb[b], en_out.at[c, pl.ds(0, K), :],
                                  semw[b]).wait()

        @pl.when(s < NB)
        def _():
            pltpu.sync_copy(zb, acc.at[pl.ds(s * NBS, NBS), :])
        plsc.subcore_barrier()

        issue_idx(0, 0)
        wait_idx(0)
        issue_gather(0, 0)
        issue_idx(1, 1)

        def do_chunk(i, b):
            srows, drows, ecb, enb = srb[b], drb[b], ecbb[b], enbb[b]
            wait_gather(b)

            @pl.when(i + 1 < CPT)
            def _():
                wait_idx(1 - b)
                issue_gather(i + 1, 1 - b)

            if write_enew:
                @pl.when(i >= 2)
                def _():
                    wait_enew(b)

            @plsc.parallel_loop(0, K, 1, unroll=4)
            def edge(j):
                for v in range(HH // 16):
                    sl = pl.ds(16 * v, 16)
                    sl2 = pl.ds(HH + 16 * v, 16)
                    slc = pl.ds(c * HH + 16 * v, 16)
                    en = srows[j, sl] + drows[j, slc] + ecb[j, slc]
                    sg = 1.0 / (1.0 + jnp.exp(-en))
                    outb[j, sl] = sg * srows[j, sl2]
                    outb[j, sl2] = sg
                    if write_enew:
                        enb[j, slc] = en

            pltpu.sync_copy(outb, acc.at[dib[b]], add=True)
            if write_enew:
                base = tbase + i * K
                pltpu.async_copy(enb, en_out.at[c, pl.ds(base, K), :],
                                 semw[b])

            @pl.when(i + 2 < CPT)
            def _():
                issue_idx(i + 2, b)

        def outer(g, carry):
            for b in (0, 1):
                do_chunk(g * 2 + b, b)
            return carry

        lax.fori_loop(0, CPT // 2, outer, 0)
        if write_enew:
            wait_enew(0)
            wait_enew(1)
        plsc.subcore_barrier()

        @pl.when(s < NB)
        def _():
            rsl = pl.ds(s * NBS, NBS)
            pltpu.sync_copy(acc.at[rsl, :], nd_out.at[c, rsl, :])

    return edge_pass


_make_edge_pass = functools.lru_cache(maxsize=None)(_make_edge_pass)


def _edge_pass_a(*args):
    return _make_edge_pass(True)(*args)


def _edge_pass_b(*args):
    return _make_edge_pass(False)(*args)


def kernel(h, e, edge_index, a, Wh, We, A1, B1, C1, D1, E1,
           A2, B2, C2, D2, E2, Wm1, bm1, Wm2, bm2):
    src = edge_index[0]
    dst = edge_index[1]
    zb = jnp.zeros((NBS, H), _F32)

    h0, stab1, dtab1, hD1 = _node_pre(h, Wh, A1, B1, E1, D1)
    ec1 = _edge_lin(e, We, C1)

    nd1, enew1 = _edge_pass_a(src, dst, stab1, dtab1, ec1, zb)

    hn1, st1 = _upd_stats(hD1, nd1)
    h1, stab2, dtab2, hD2 = _h1_pre2(h0, hn1, st1, A2, B2, E2, D2)

    est = _estats(enew1)
    ec2 = _edge2(enew1, e, est, We, C2)

    nd2 = _edge_pass_b(src, dst, stab2, dtab2, ec2, zb)

    hn2, st2 = _upd_stats(hD2, nd2)
    return _final(h1, hn2, st2, a, Wm1, bm1.reshape(1, MLP_H),
                  Wm2.reshape(1, MLP_H), bm2.reshape(1, 1))


# speedup vs baseline: 1.0035x; 1.0035x over previous
import functools

import jax
import jax.numpy as jnp
from jax import lax
from jax.experimental import pallas as pl
from jax.experimental.pallas import tpu as pltpu
from jax.experimental.pallas import tpu_sc as plsc

N = 10000
E = 320000
H = 128
HH = 64
A_DIM = 8
MLP_H = 64

NB = 10
NBS = N // NB
EBLK = 80
EBS = E // EBLK

NC = 2
NT = 16
K = 40
CPT = E // (K * NT)

EPS_DEN = 1e-6
EPS_BN = 1e-5

_F32 = jnp.float32


def _dot(a, b):
    return jnp.dot(a, b, preferred_element_type=_F32)


def _node_pre_body(h_ref, wh, a1, b1, e1w, d1, h0_o, st_o, dt_o, hd_o):
    h0 = _dot(h_ref[...], wh[...])
    h0_o[...] = h0
    ha = _dot(h0, a1[...])
    hv = _dot(h0, e1w[...])
    st_o[0] = jnp.concatenate([ha[:, :HH], hv[:, :HH]], axis=1)
    st_o[1] = jnp.concatenate([ha[:, HH:], hv[:, HH:]], axis=1)
    dt_o[...] = _dot(h0, b1[...])
    hd_o[...] = _dot(h0, d1[...])


def _edge_lin_body(e_ref, we_ref, c_ref, out_ref):
    wec = _dot(we_ref[...], c_ref[...])
    out_ref[...] = _dot(e_ref[...], wec)


def _upd_stats_body(hd_ref, nd_ref, hn_o, st_o):
    num = jnp.concatenate([nd_ref[0, :, :HH], nd_ref[1, :, :HH]], axis=1)
    den = jnp.concatenate([nd_ref[0, :, HH:], nd_ref[1, :, HH:]], axis=1)
    hn = hd_ref[...] + num / (den + EPS_DEN)
    hn_o[...] = hn

    @pl.when(pl.program_id(0) == 0)
    def _():
        st_o[...] = jnp.zeros_like(st_o)

    part = jnp.concatenate(
        [jnp.sum(hn, 0, keepdims=True), jnp.sum(hn * hn, 0, keepdims=True)], 0)
    st_o[0:2, :] += part


def _h1_pre2_body(hres_ref, hn_ref, st_ref, a2, b2, e2w, d2,
                  h1_o, st_o, dt_o, hd_o):
    m = st_ref[0:1, :] / N
    var = st_ref[1:2, :] / N - m * m
    rstd = lax.rsqrt(var + EPS_BN)
    h1 = hres_ref[...] + jnp.maximum((hn_ref[...] - m) * rstd, 0.0)
    h1_o[...] = h1
    ha = _dot(h1, a2[...])
    hv = _dot(h1, e2w[...])
    st_o[0] = jnp.concatenate([ha[:, :HH], hv[:, :HH]], axis=1)
    st_o[1] = jnp.concatenate([ha[:, HH:], hv[:, HH:]], axis=1)
    dt_o[...] = _dot(h1, b2[...])
    hd_o[...] = _dot(h1, d2[...])


def _estats_body(en_ref, st_o):
    x = jnp.concatenate([en_ref[0][:, :HH], en_ref[1][:, HH:]], axis=1)
    s = jnp.sum(x, 0, keepdims=True)
    q = jnp.sum(x * x, 0, keepdims=True)

    @pl.when(pl.program_id(0) == 0)
    def _():
        st_o[...] = jnp.zeros_like(st_o)

    st_o[0:2, :] += jnp.concatenate([s, q], 0)


def _edge2_body(en_ref, e_ref, st_ref, we_ref, c2_ref, out_ref):
    m = st_ref[0:1, :] / E
    var = st_ref[1:2, :] / E - m * m
    rstd = lax.rsqrt(var + EPS_BN)
    en = jnp.concatenate([en_ref[0][:, :HH], en_ref[1][:, HH:]], axis=1)
    r = jnp.maximum((en - m) * rstd, 0.0)
    wec2 = _dot(we_ref[...], c2_ref[...])
    out_ref[...] = _dot(r, c2_ref[...]) + _dot(e_ref[...], wec2)


def _final_body(h1_ref, hn_ref, st_ref, a_ref, wm1, bm1, wm2t, bm2,
                out_ref, acc):
    m = st_ref[0:1, :] / N
    var = st_ref[1:2, :] / N - m * m
    rstd = lax.rsqrt(var + EPS_BN)
    h2 = h1_ref[...] + jnp.maximum((hn_ref[...] - m) * rstd, 0.0)
    cs = jnp.sum(h2, 0, keepdims=True)
    pid = pl.program_id(0)

    @pl.when(pid == 0)
    def _():
        acc[...] = jnp.zeros_like(acc)

    acc[0:1, :] += cs

    @pl.when(pid == NB - 1)
    def _():
        hg = acc[0:1, :] / N
        z = jnp.concatenate([hg, a_ref[...]], axis=1)
        q = jnp.maximum(_dot(z, wm1[...]) + bm1[...], 0.0)
        out_ref[...] = jnp.sum(q * wm2t[...], axis=1, keepdims=True) + bm2[...]


def _full(shape):
    return pl.BlockSpec(shape, lambda i: tuple(0 for _ in shape))


def _node_pre(h, Wh, A1, B1, E1, D1):
    row = pl.BlockSpec((NBS, H), lambda i: (i, 0))
    srow = pl.BlockSpec((NC, NBS, H), lambda i: (0, i, 0))
    w = _full((H, H))
    return pl.pallas_call(
        _node_pre_body,
        grid=(NB,),
        in_specs=[row, w, w, w, w, w],
        out_specs=[row, srow, row, row],
        out_shape=[jax.ShapeDtypeStruct((N, H), _F32),
                   jax.ShapeDtypeStruct((NC, N, H), _F32),
                   jax.ShapeDtypeStruct((N, H), _F32),
                   jax.ShapeDtypeStruct((N, H), _F32)],
    )(h, Wh, A1, B1, E1, D1)


def _edge_lin(e, We, C1):
    return pl.pallas_call(
        _edge_lin_body,
        grid=(EBLK,),
        in_specs=[pl.BlockSpec((EBS, 16), lambda i: (i, 0)),
                  _full((16, H)), _full((H, H))],
        out_specs=pl.BlockSpec((EBS, H), lambda i: (i, 0)),
        out_shape=jax.ShapeDtypeStruct((E, H), _F32),
    )(e, We, C1)


def _upd_stats(hd, nd):
    row = pl.BlockSpec((NBS, H), lambda i: (i, 0))
    return pl.pallas_call(
        _upd_stats_body,
        grid=(NB,),
        in_specs=[row, pl.BlockSpec((NC, NBS, H), lambda i: (0, i, 0))],
        out_specs=[row, _full((8, H))],
        out_shape=[jax.ShapeDtypeStruct((N, H), _F32),
                   jax.ShapeDtypeStruct((8, H), _F32)],
    )(hd, nd)


def _h1_pre2(h0, hn1, st1, A2, B2, E2, D2):
    row = pl.BlockSpec((NBS, H), lambda i: (i, 0))
    srow = pl.BlockSpec((NC, NBS, H), lambda i: (0, i, 0))
    w = _full((H, H))
    return pl.pallas_call(
        _h1_pre2_body,
        grid=(NB,),
        in_specs=[row, row, _full((8, H)), w, w, w, w],
        out_specs=[row, srow, row, row],
        out_shape=[jax.ShapeDtypeStruct((N, H), _F32),
                   jax.ShapeDtypeStruct((NC, N, H), _F32),
                   jax.ShapeDtypeStruct((N, H), _F32),
                   jax.ShapeDtypeStruct((N, H), _F32)],
    )(h0, hn1, st1, A2, B2, E2, D2)


def _estats(enew):
    return pl.pallas_call(
        _estats_body,
        grid=(EBLK,),
        in_specs=[pl.BlockSpec((NC, EBS, H), lambda i: (0, i, 0))],
        out_specs=_full((8, H)),
        out_shape=jax.ShapeDtypeStruct((8, H), _F32),
    )(enew)


def _edge2(enew, e, est, We, C2):
    return pl.pallas_call(
        _edge2_body,
        grid=(EBLK,),
        in_specs=[pl.BlockSpec((NC, EBS, H), lambda i: (0, i, 0)),
                  pl.BlockSpec((EBS, 16), lambda i: (i, 0)),
                  _full((8, H)), _full((16, H)), _full((H, H))],
        out_specs=pl.BlockSpec((EBS, H), lambda i: (i, 0)),
        out_shape=jax.ShapeDtypeStruct((E, H), _F32),
    )(enew, e, est, We, C2)


def _final(h1, hn2, st2, a, Wm1, bm1, Wm2, bm2):
    row = pl.BlockSpec((NBS, H), lambda i: (i, 0))
    return pl.pallas_call(
        _final_body,
        grid=(NB,),
        in_specs=[row, row, _full((8, H)), _full((1, A_DIM)),
                  _full((H + A_DIM, MLP_H)), _full((1, MLP_H)),
                  _full((1, MLP_H)), _full((1, 1))],
        out_specs=_full((1, 1)),
        out_shape=jax.ShapeDtypeStruct((1, 1), _F32),
        scratch_shapes=[pltpu.VMEM((8, H), _F32)],
    )(h1, hn2, st2, a, Wm1, bm1, Wm2, bm2)


def _make_edge_pass(write_enew):
    mesh = plsc.VectorSubcoreMesh(core_axis_name="c", subcore_axis_name="s",
                                  num_cores=NC, num_subcores=NT)
    outs = [jax.ShapeDtypeStruct((NC, N, H), _F32)]
    if write_enew:
        outs.append(jax.ShapeDtypeStruct((NC, E, H), _F32))
    scratch = [
        pltpu.VMEM((K,), jnp.int32),
        pltpu.VMEM((K,), jnp.int32),
        pltpu.VMEM((K,), jnp.int32),
        pltpu.VMEM((K,), jnp.int32),
        pltpu.VMEM((K, H), _F32),
        pltpu.VMEM((K, H), _F32),
        pltpu.VMEM((K, H), _F32),
        pltpu.VMEM((K, H), _F32),
        pltpu.VMEM((K, H), _F32),
        pltpu.VMEM((K, H), _F32),
        pltpu.VMEM((K, H), _F32),
        pltpu.VMEM((K, H), _F32),
        pltpu.VMEM((K, H), _F32),
        pltpu.SemaphoreType.DMA,
        pltpu.SemaphoreType.DMA,
        pltpu.SemaphoreType.DMA,
        pltpu.SemaphoreType.DMA,
        pltpu.SemaphoreType.DMA,
        pltpu.SemaphoreType.DMA,
        pltpu.VMEM_SHARED((N, H), _F32),
    ]

    @functools.partial(pl.kernel, mesh=mesh,
                       out_type=tuple(outs) if write_enew else outs[0],
                       scratch_types=scratch)
    def edge_pass(ei_hbm, stab, dtab, ec, zb, *refs):
        if write_enew:
            nd_out, en_out = refs[0], refs[1]
            scr = refs[2:]
        else:
            nd_out = refs[0]
            en_out = None
            scr = refs[1:]
        (si0, di0, si1, di1, sr0, sr1, dr0, dr1, eb0, eb1, outb, en0, en1,
         semi0, semi1, semg0, semg1, semw0, semw1, acc) = scr
        sib, dib = (si0, si1), (di0, di1)
        srb, drb, ecbb = (sr0, sr1), (dr0, dr1), (eb0, eb1)
        enbb = (en0, en1)
        semi, semg, semw = (semi0, semi1), (semg0, semg1), (semw0, semw1)
        c = lax.axis_index("c")
        s = lax.axis_index("s")
        tbase = s * (CPT * K)

        def issue_idx(i, b):
            base = tbase + i * K
            pltpu.async_copy(ei_hbm.at[pl.ds(base, K)], sib[b], semi[b])
            pltpu.async_copy(ei_hbm.at[pl.ds(E + base, K)], dib[b], semi[b])

        def wait_idx(b):
            pltpu.make_async_copy(ei_hbm.at[pl.ds(0, K)], sib[b],
                                  semi[b]).wait()
            pltpu.make_async_copy(ei_hbm.at[pl.ds(0, K)], dib[b],
                                  semi[b]).wait()

        def issue_gather(i, b):
            base = tbase + i * K
            pltpu.async_copy(stab.at[c].at[sib[b]], srb[b], semg[b])
            pltpu.async_copy(dtab.at[dib[b]], drb[b], semg[b])
            pltpu.async_copy(ec.at[pl.ds(base, K), :], ecbb[b], semg[b])

        def wait_gather(b):
            pltpu.make_async_copy(stab.at[c].at[sib[b]], srb[b],
                                  semg[b]).wait()
            pltpu.make_async_copy(dtab.at[dib[b]], drb[b],
                                  semg[b]).wait()
            pltpu.make_async_copy(ec.at[pl.ds(0, K), :], ecbb[b],
                                  semg[b]).wait()

        def wait_enew(b):
            pltpu.make_async_copy(enbb[b], en_out.at[c, pl.ds(0, K), :],
                                  semw[b]).wait()

        @pl.when(s < NB)
        def _():
            pltpu.sync_copy(zb, acc.at[pl.ds(s * NBS, NBS), :])
        plsc.subcore_barrier()

        issue_idx(0, 0)
        wait_idx(0)
        issue_gather(0, 0)
        issue_idx(1, 1)

        def do_chunk(i, b):
            srows, drows, ecb, enb = srb[b], drb[b], ecbb[b], enbb[b]
            wait_gather(b)

            @pl.when(i + 1 < CPT)
            def _():
                wait_idx(1 - b)
                issue_gather(i + 1, 1 - b)

            if write_enew:
                @pl.when(i >= 2)
                def _():
                    wait_enew(b)

            @plsc.parallel_loop(0, K, 1, unroll=2)
            def edge(j):
                for v in range(HH // 16):
                    sl = pl.ds(16 * v, 16)
                    sl2 = pl.ds(HH + 16 * v, 16)
                    slc = pl.ds(c * HH + 16 * v, 16)
                    en = srows[j, sl] + drows[j, slc] + ecb[j, slc]
                    sg = 1.0 / (1.0 + jnp.exp(-en))
                    outb[j, sl] = sg * srows[j, sl2]
                    outb[j, sl2] = sg
                    if write_enew:
                        enb[j, slc] = en

            pltpu.sync_copy(outb, acc.at[dib[b]], add=True)
            if write_enew:
                base = tbase + i * K
                pltpu.async_copy(enb, en_out.at[c, pl.ds(base, K), :],
                                 semw[b])

            @pl.when(i + 2 < CPT)
            def _():
                issue_idx(i + 2, b)

        def outer(g, carry):
            for b in (0, 1):
                do_chunk(g * 2 + b, b)
            return carry

        lax.fori_loop(0, CPT // 2, outer, 0)
        if write_enew:
            wait_enew(0)
            wait_enew(1)
        plsc.subcore_barrier()

        @pl.when(s < NB)
        def _():
            rsl = pl.ds(s * NBS, NBS)
            pltpu.sync_copy(acc.at[rsl, :], nd_out.at[c, rsl, :])

    return edge_pass


_make_edge_pass = functools.lru_cache(maxsize=None)(_make_edge_pass)


def _edge_pass_a(*args):
    return _make_edge_pass(True)(*args)


def _edge_pass_b(*args):
    return _make_edge_pass(False)(*args)


def kernel(h, e, edge_index, a, Wh, We, A1, B1, C1, D1, E1,
           A2, B2, C2, D2, E2, Wm1, bm1, Wm2, bm2):
    zb = jnp.zeros((NBS, H), _F32)

    h0, stab1, dtab1, hD1 = _node_pre(h, Wh, A1, B1, E1, D1)
    ec1 = _edge_lin(e, We, C1)

    nd1, enew1 = _edge_pass_a(edge_index.reshape(-1), stab1, dtab1, ec1, zb)

    hn1, st1 = _upd_stats(hD1, nd1)
    h1, stab2, dtab2, hD2 = _h1_pre2(h0, hn1, st1, A2, B2, E2, D2)

    est = _estats(enew1)
    ec2 = _edge2(enew1, e, est, We, C2)

    nd2 = _edge_pass_b(edge_index.reshape(-1), stab2, dtab2, ec2, zb)

    hn2, st2 = _upd_stats(hD2, nd2)
    return _final(h1, hn2, st2, a, Wm1, bm1.reshape(1, MLP_H),
                  Wm2.reshape(1, MLP_H), bm2.reshape(1, 1))
